# Initial kernel scaffold; baseline (speedup 1.0000x reference)
import jax
import jax.numpy as jnp
from jax.experimental import pallas as pl


def kernel(c_in, c_p, bandwidth):
    raise NotImplementedError("write your pallas kernel here")


# speedup vs baseline: 60.8364x; 60.8364x over previous
import functools

import jax
import jax.numpy as jnp
from jax import lax
from jax.experimental import pallas as pl
from jax.experimental.pallas import tpu as pltpu
from jax.experimental.pallas import tpu_sc as plsc

_L = 16


def _lane_permute(v, idx):
    dnums = lax.GatherDimensionNumbers(
        offset_dims=(), collapsed_slice_dims=(0,), start_index_map=(0,))
    return lax.gather(v, idx[:, None], dnums, (1,),
                      mode=lax.GatherScatterMode.PROMISE_IN_BOUNDS)


def _lane_max(v):
    for sh in (8, 4, 2, 1):
        idx = lax.iota(jnp.int32, _L) ^ sh
        v = jnp.maximum(v, _lane_permute(v, idx))
    return v


def _lane_sum(v):
    for sh in (8, 4, 2, 1):
        idx = lax.iota(jnp.int32, _L) ^ sh
        v = v + _lane_permute(v, idx)
    return v


def _make_sc_kernel(N, F, B, K):
    info = plsc.get_sparse_core_info()
    NC, NS = info.num_cores, info.num_subcores
    assert NC * NS == B, (NC, NS, B)
    n_chunks = N // _L
    k_vecs = K // _L
    mesh = plsc.VectorSubcoreMesh(core_axis_name="c", subcore_axis_name="s")

    def body(cp_hbm, ct_hbm, bw_hbm, out_hbm, tab_v, idx_v, w_v, bw_v):
        b = lax.axis_index("s") * NC + lax.axis_index("c")
        pltpu.sync_copy(cp_hbm.at[b], tab_v)
        pltpu.sync_copy(ct_hbm, idx_v)
        pltpu.sync_copy(bw_hbm, bw_v)

        c_acc = jnp.zeros((_L,), jnp.float32)
        for i in range(F):
            base = i * K
            m = tab_v[pl.ds(base, _L)]
            for j in range(1, k_vecs):
                m = jnp.maximum(m, tab_v[pl.ds(base + j * _L, _L)])
            m_b = _lane_max(m)
            se = jnp.zeros((_L,), jnp.float32)
            se2 = jnp.zeros((_L,), jnp.float32)
            for j in range(k_vecs):
                e = jnp.exp(tab_v[pl.ds(base + j * _L, _L)] - m_b)
                tab_v[pl.ds(base + j * _L, _L)] = e
                se = se + e
                se2 = se2 + e * e
            inv_b = jnp.float32(1.0) / _lane_sum(se)
            for j in range(k_vecs):
                tab_v[pl.ds(base + j * _L, _L)] = (
                    tab_v[pl.ds(base + j * _L, _L)] * inv_b
                )
            c_acc = c_acc + _lane_sum(se2) * inv_b * inv_b + jnp.float32(1.0)

        inv_bw = jnp.float32(1.0) / jnp.clip(bw_v[...], 0.1, 10.0)
        a_v = jnp.float32(2.0) * inv_bw
        c_v = c_acc * inv_bw

        def chunk(j, sacc):
            off = pl.multiple_of(j * _L, _L)
            g = plsc.load_gather(tab_v, [idx_v[0, pl.ds(off, _L)]])
            for i in range(1, F):
                g = g + plsc.load_gather(
                    tab_v, [idx_v[i, pl.ds(off, _L)] + jnp.int32(i * K)]
                )
            w = jnp.exp(g * a_v - c_v)
            w_v[pl.ds(off, _L)] = w
            return sacc + w

        sacc = lax.fori_loop(0, n_chunks, chunk,
                             jnp.zeros((_L,), jnp.float32))
        s_b = _lane_sum(sacc)
        norm_b = jnp.where(s_b < jnp.float32(1e-13),
                           jnp.float32(0.0), jnp.float32(1.0) / s_b)

        def nrm(j, carry):
            off = pl.multiple_of(j * _L, _L)
            w_v[pl.ds(off, _L)] = w_v[pl.ds(off, _L)] * norm_b
            return carry

        lax.fori_loop(0, n_chunks, nrm, jnp.int32(0))
        pltpu.sync_copy(w_v, out_hbm.at[b])

    return pl.kernel(
        body,
        out_type=jax.ShapeDtypeStruct((B, N), jnp.float32),
        mesh=mesh,
        compiler_params=pltpu.CompilerParams(needs_layout_passes=False),
        scratch_types=[
            pltpu.VMEM((F * K,), jnp.float32),
            pltpu.VMEM((F, N), jnp.int32),
            pltpu.VMEM((N,), jnp.float32),
            pltpu.VMEM((_L,), jnp.float32),
        ],
    )


@jax.jit
def kernel(c_in, c_p, bandwidth):
    F, B, K = c_p.shape
    N = c_in.shape[0]
    cp_t = jnp.transpose(c_p, (1, 0, 2)).reshape(B, F * K)
    ct = c_in.T.astype(jnp.int32)
    bw16 = jnp.broadcast_to(bandwidth.astype(jnp.float32), (_L,))
    out = _make_sc_kernel(N, F, B, K)(cp_t, ct, bw16)
    return out[:, :, None]
